# baseline (speedup 1.0000x reference)
import functools

import jax
import jax.numpy as jnp
from jax import lax
from jax.experimental import pallas as pl
from jax.experimental.pallas import tpu as pltpu
from jax.experimental.pallas import tpu_sc as plsc


@functools.cache
def _make_sc_gather_softmax(n_neurons: int, n_sf: int, b: int):
    info = plsc.get_sparse_core_info()
    nc, ns, nl = info.num_cores, info.num_subcores, info.num_lanes
    nw = nc * ns
    b_per_w = b // nw
    nv = n_sf // nl
    mesh = plsc.VectorSubcoreMesh(core_axis_name="c", subcore_axis_name="s")

    @functools.partial(
        pl.kernel,
        mesh=mesh,
        out_type=jax.ShapeDtypeStruct((b, n_sf), jnp.float32),
        scratch_types=[
            pltpu.VMEM((b_per_w,), jnp.int32),
            pltpu.VMEM((b_per_w, n_sf), jnp.float32),
            pltpu.SemaphoreType.DMA,
        ],
    )
    def sc_kernel(table_hbm, idx_hbm, out_hbm, idx_v, rows_v, sem):
        wid = lax.axis_index("s") * nc + lax.axis_index("c")
        base = wid * b_per_w
        pltpu.sync_copy(idx_hbm.at[pl.ds(base, b_per_w)], idx_v)
        pltpu.async_copy(table_hbm.at[idx_v], rows_v, sem).wait()

        lane = lax.iota(jnp.int32, nl)

        gather_dn = lax.GatherDimensionNumbers(
            offset_dims=(), collapsed_slice_dims=(0,), start_index_map=(0,))

        def shuffle(v, sh):
            return lax.gather(v, (lane ^ sh)[:, None], gather_dn,
                              slice_sizes=(1,),
                              mode=lax.GatherScatterMode.PROMISE_IN_BOUNDS)

        def butterfly(v, op):
            sh = nl // 2
            while sh:
                v = op(v, shuffle(v, sh))
                sh //= 2
            return v

        for r in range(b_per_w):
            vs = [rows_v[r, pl.ds(j * nl, nl)] for j in range(nv)]
            m = vs[0]
            for j in range(1, nv):
                m = jnp.maximum(m, vs[j])
            row_max = butterfly(m, jnp.maximum)
            es = [jnp.exp(v - row_max) for v in vs]
            acc = es[0]
            for j in range(1, nv):
                acc = acc + es[j]
            inv = 1.0 / butterfly(acc, jnp.add)
            for j in range(nv):
                rows_v[r, pl.ds(j * nl, nl)] = es[j] * inv
        pltpu.sync_copy(rows_v, out_hbm.at[pl.ds(base, b_per_w)])

    return sc_kernel


def _tc_weighted_sum(xr, w):
    bsz, n_sf, cs = xr.shape
    bb = 8

    def body(x_ref, w_ref, o_ref):
        o_ref[...] = jnp.sum(x_ref[...] * w_ref[...][:, :, None], axis=1)

    return pl.pallas_call(
        body,
        grid=(bsz // bb,),
        in_specs=[
            pl.BlockSpec((bb, n_sf, cs), lambda i: (i, 0, 0)),
            pl.BlockSpec((bb, n_sf), lambda i: (i, 0)),
        ],
        out_specs=pl.BlockSpec((bb, cs), lambda i: (i, 0)),
        out_shape=jax.ShapeDtypeStruct((bsz, cs), jnp.float32),
    )(xr, w)


def kernel(x, neuron_list, alphas):
    b, n, c, s = x.shape
    xr = x.reshape(b, n, c * s)
    n_neurons, n_sf = alphas.shape
    alphas_att = _make_sc_gather_softmax(n_neurons, n_sf, b)(alphas, neuron_list)
    out = _tc_weighted_sum(xr, alphas_att)
    return out, alphas_att


# speedup vs baseline: 1.1365x; 1.1365x over previous
import functools

import jax
import jax.numpy as jnp
from jax import lax
from jax.experimental import pallas as pl
from jax.experimental.pallas import tpu as pltpu
from jax.experimental.pallas import tpu_sc as plsc


@functools.cache
def _make_sc_gather_softmax(n_neurons: int, n_sf: int, b: int):
    info = plsc.get_sparse_core_info()
    nc, ns, nl = info.num_cores, info.num_subcores, info.num_lanes
    nw = nc * ns
    b_per_w = b // nw
    nv = n_sf // nl
    mesh = plsc.VectorSubcoreMesh(core_axis_name="c", subcore_axis_name="s")

    @functools.partial(
        pl.kernel,
        mesh=mesh,
        out_type=jax.ShapeDtypeStruct((b, n_sf), jnp.float32),
        scratch_types=[
            pltpu.VMEM((b_per_w,), jnp.int32),
            pltpu.VMEM((b_per_w, n_sf), jnp.float32),
            pltpu.SemaphoreType.DMA,
        ],
    )
    def sc_kernel(table_hbm, idx_hbm, out_hbm, idx_v, rows_v, sem):
        wid = lax.axis_index("s") * nc + lax.axis_index("c")
        base = wid * b_per_w
        pltpu.sync_copy(idx_hbm.at[pl.ds(base, b_per_w)], idx_v)
        pltpu.async_copy(table_hbm.at[idx_v], rows_v, sem).wait()

        lane = lax.iota(jnp.int32, nl)

        gather_dn = lax.GatherDimensionNumbers(
            offset_dims=(), collapsed_slice_dims=(0,), start_index_map=(0,))

        def shuffle(v, sh):
            return lax.gather(v, (lane ^ sh)[:, None], gather_dn,
                              slice_sizes=(1,),
                              mode=lax.GatherScatterMode.PROMISE_IN_BOUNDS)

        def butterfly(v, op):
            sh = nl // 2
            while sh:
                v = op(v, shuffle(v, sh))
                sh //= 2
            return v

        for r in range(b_per_w):
            vs = [rows_v[r, pl.ds(j * nl, nl)] for j in range(nv)]
            m = vs[0]
            for j in range(1, nv):
                m = jnp.maximum(m, vs[j])
            row_max = butterfly(m, jnp.maximum)
            es = [jnp.exp(v - row_max) for v in vs]
            acc = es[0]
            for j in range(1, nv):
                acc = acc + es[j]
            inv = 1.0 / butterfly(acc, jnp.add)
            for j in range(nv):
                rows_v[r, pl.ds(j * nl, nl)] = es[j] * inv
        pltpu.sync_copy(rows_v, out_hbm.at[pl.ds(base, b_per_w)])

    return sc_kernel


def _tc_weighted_sum(xr, w):
    bsz, n_sf, cs = xr.shape
    bb = 32

    def body(x_ref, w_ref, o_ref):
        o_ref[...] = jnp.sum(x_ref[...] * w_ref[...][:, :, None], axis=1)

    return pl.pallas_call(
        body,
        grid=(bsz // bb,),
        in_specs=[
            pl.BlockSpec((bb, n_sf, cs), lambda i: (i, 0, 0)),
            pl.BlockSpec((bb, n_sf), lambda i: (i, 0)),
        ],
        out_specs=pl.BlockSpec((bb, cs), lambda i: (i, 0)),
        out_shape=jax.ShapeDtypeStruct((bsz, cs), jnp.float32),
    )(xr, w)


def kernel(x, neuron_list, alphas):
    b, n, c, s = x.shape
    xr = x.reshape(b, n, c * s)
    n_neurons, n_sf = alphas.shape
    alphas_att = _make_sc_gather_softmax(n_neurons, n_sf, b)(alphas, neuron_list)
    out = _tc_weighted_sum(xr, alphas_att)
    return out, alphas_att
